# Initial kernel scaffold; baseline (speedup 1.0000x reference)
import jax
import jax.numpy as jnp
from jax.experimental import pallas as pl


def kernel(x, table, W1, b1, W2, b2):
    raise NotImplementedError("write your pallas kernel here")


# speedup vs baseline: 8.9010x; 8.9010x over previous
import functools

import jax
import jax.numpy as jnp
from jax import lax
from jax.experimental import pallas as pl
from jax.experimental.pallas import tpu as pltpu
from jax.experimental.pallas import tpu_sc as plsc

BATCH = 16384
SEQ = 200
EMBED = 32
HIDDEN = 128
CLS = 10

NC, NS = 2, 16
NW = NC * NS
ROWS_PER_W = BATCH // NW
CHUNK = 32
NCHUNK = ROWS_PER_W // CHUNK
SEQ_A = 128
SEQ_B = SEQ - SEQ_A
INV_SEQ = 1.0 / SEQ


def _pool_sc(x, table):
    mesh = plsc.VectorSubcoreMesh(core_axis_name="c", subcore_axis_name="s")

    @functools.partial(
        pl.kernel,
        out_type=jax.ShapeDtypeStruct((BATCH, EMBED), jnp.float32),
        mesh=mesh,
        scratch_types=[
            pltpu.VMEM((CHUNK, SEQ), jnp.int32),
            pltpu.VMEM((SEQ, EMBED), jnp.float32),
            pltpu.VMEM((CHUNK, EMBED), jnp.float32),
            pltpu.SemaphoreType.DMA,
            pltpu.SemaphoreType.DMA,
        ],
        compiler_params=pltpu.CompilerParams(use_tc_tiling_on_sc=False),
    )
    def k(x_hbm, tab_hbm, out_hbm, idx_v, g_v, o_v, sem_a, sem_b):
        wid = lax.axis_index("s") * NC + lax.axis_index("c")
        base = wid * ROWS_PER_W

        @pl.loop(0, NCHUNK)
        def _chunk(ci):
            cbase = base + ci * CHUNK
            pltpu.sync_copy(x_hbm.at[pl.ds(cbase, CHUNK)], idx_v)

            @pl.loop(0, CHUNK)
            def _row(r):
                ca = pltpu.async_copy(
                    tab_hbm.at[idx_v.at[r, pl.ds(0, SEQ_A)]],
                    g_v.at[pl.ds(0, SEQ_A)], sem_a)
                cb = pltpu.async_copy(
                    tab_hbm.at[idx_v.at[r, pl.ds(SEQ_A, SEQ_B)]],
                    g_v.at[pl.ds(SEQ_A, SEQ_B)], sem_b)
                ca.wait()
                cb.wait()

                def body(i, carry):
                    a0, a1 = carry
                    return (a0 + g_v[i, pl.ds(0, 16)],
                            a1 + g_v[i, pl.ds(16, 16)])

                a0, a1 = lax.fori_loop(
                    0, SEQ, body,
                    (jnp.zeros((16,), jnp.float32),
                     jnp.zeros((16,), jnp.float32)))
                o_v[r, pl.ds(0, 16)] = a0 * INV_SEQ
                o_v[r, pl.ds(16, 16)] = a1 * INV_SEQ

            pltpu.sync_copy(o_v, out_hbm.at[pl.ds(cbase, CHUNK)])

    return k(x, table)


def _mlp_tc(pooled, W1, b1, W2, b2):
    BB = 2048

    def body(p_ref, w1_ref, b1_ref, w2_ref, b2_ref, o_ref):
        h = jnp.dot(p_ref[...], w1_ref[...],
                    preferred_element_type=jnp.float32)
        h = jnp.maximum(h + b1_ref[...], 0.0)
        o_ref[...] = jnp.dot(h, w2_ref[...],
                             preferred_element_type=jnp.float32) + b2_ref[...]

    return pl.pallas_call(
        body,
        grid=(BATCH // BB,),
        in_specs=[
            pl.BlockSpec((BB, EMBED), lambda i: (i, 0)),
            pl.BlockSpec((EMBED, HIDDEN), lambda i: (0, 0)),
            pl.BlockSpec((1, HIDDEN), lambda i: (0, 0)),
            pl.BlockSpec((HIDDEN, CLS), lambda i: (0, 0)),
            pl.BlockSpec((1, CLS), lambda i: (0, 0)),
        ],
        out_specs=pl.BlockSpec((BB, CLS), lambda i: (i, 0)),
        out_shape=jax.ShapeDtypeStruct((BATCH, CLS), jnp.float32),
    )(pooled, W1, b1.reshape(1, HIDDEN), W2, b2.reshape(1, CLS))


def kernel(x, table, W1, b1, W2, b2):
    pooled = _pool_sc(x, table)
    return _mlp_tc(pooled, W1, b1, W2, b2)


# speedup vs baseline: 16.3622x; 1.8382x over previous
import functools

import jax
import jax.numpy as jnp
from jax import lax
from jax.experimental import pallas as pl
from jax.experimental.pallas import tpu as pltpu
from jax.experimental.pallas import tpu_sc as plsc

BATCH = 16384
SEQ = 200
EMBED = 32
HIDDEN = 128
CLS = 10

NC, NS = 2, 16
NW = NC * NS
ROWS_PER_W = BATCH // NW
HALF = 256
SEQ_A = 128
SEQ_B = SEQ - SEQ_A
NBUF = 4
INV_SEQ = 1.0 / SEQ


def _pool_sc(x, table):
    mesh = plsc.VectorSubcoreMesh(core_axis_name="c", subcore_axis_name="s")

    @functools.partial(
        pl.kernel,
        out_type=jax.ShapeDtypeStruct((BATCH, EMBED), jnp.float32),
        mesh=mesh,
        scratch_types=[
            pltpu.VMEM((HALF, SEQ), jnp.int32),
            pltpu.VMEM((NBUF, SEQ, EMBED), jnp.float32),
            pltpu.VMEM((HALF, EMBED), jnp.float32),
            [pltpu.SemaphoreType.DMA] * NBUF,
        ],
        compiler_params=pltpu.CompilerParams(use_tc_tiling_on_sc=False),
    )
    def k(x_hbm, tab_hbm, out_hbm, idx_v, g_v, o_v, sems):
        wid = lax.axis_index("s") * NC + lax.axis_index("c")
        base = wid * ROWS_PER_W

        def issue(row, b):
            pltpu.async_copy(
                tab_hbm.at[idx_v.at[row, pl.ds(0, SEQ_A)]],
                g_v.at[b, pl.ds(0, SEQ_A)], sems[b])
            pltpu.async_copy(
                tab_hbm.at[idx_v.at[row, pl.ds(SEQ_A, SEQ_B)]],
                g_v.at[b, pl.ds(SEQ_A, SEQ_B)], sems[b])

        def drain(b):
            pltpu.make_async_copy(
                tab_hbm.at[idx_v.at[0, pl.ds(0, SEQ_A)]],
                g_v.at[b, pl.ds(0, SEQ_A)], sems[b]).wait()
            pltpu.make_async_copy(
                tab_hbm.at[idx_v.at[0, pl.ds(SEQ_A, SEQ_B)]],
                g_v.at[b, pl.ds(SEQ_A, SEQ_B)], sems[b]).wait()

        for half in range(ROWS_PER_W // HALF):
            hbase = base + half * HALF
            pltpu.sync_copy(x_hbm.at[pl.ds(hbase, HALF)], idx_v)
            for b in range(NBUF):
                issue(b, b)

            @pl.loop(0, HALF, step=NBUF)
            def _rows(rc):
                for b in range(NBUF):
                    r = rc + b
                    drain(b)

                    def body(i, carry):
                        a0, a1 = carry
                        return (a0 + g_v[b, i, pl.ds(0, 16)],
                                a1 + g_v[b, i, pl.ds(16, 16)])

                    a0, a1 = lax.fori_loop(
                        0, SEQ, body,
                        (jnp.zeros((16,), jnp.float32),
                         jnp.zeros((16,), jnp.float32)),
                        unroll=8)
                    o_v[r, pl.ds(0, 16)] = a0 * INV_SEQ
                    o_v[r, pl.ds(16, 16)] = a1 * INV_SEQ

                    @pl.when(rc + NBUF < HALF)
                    def _prefetch():
                        issue(r + NBUF, b)

            pltpu.sync_copy(o_v, out_hbm.at[pl.ds(hbase, HALF)])

    return k(x, table)


def _mlp_tc(pooled, W1, b1, W2, b2):
    BB = 2048

    def body(p_ref, w1_ref, b1_ref, w2_ref, b2_ref, o_ref):
        h = jnp.dot(p_ref[...], w1_ref[...],
                    preferred_element_type=jnp.float32)
        h = jnp.maximum(h + b1_ref[...], 0.0)
        o_ref[...] = jnp.dot(h, w2_ref[...],
                             preferred_element_type=jnp.float32) + b2_ref[...]

    return pl.pallas_call(
        body,
        grid=(BATCH // BB,),
        in_specs=[
            pl.BlockSpec((BB, EMBED), lambda i: (i, 0)),
            pl.BlockSpec((EMBED, HIDDEN), lambda i: (0, 0)),
            pl.BlockSpec((1, HIDDEN), lambda i: (0, 0)),
            pl.BlockSpec((HIDDEN, CLS), lambda i: (0, 0)),
            pl.BlockSpec((1, CLS), lambda i: (0, 0)),
        ],
        out_specs=pl.BlockSpec((BB, CLS), lambda i: (i, 0)),
        out_shape=jax.ShapeDtypeStruct((BATCH, CLS), jnp.float32),
    )(pooled, W1, b1.reshape(1, HIDDEN), W2, b2.reshape(1, CLS))


def kernel(x, table, W1, b1, W2, b2):
    pooled = _pool_sc(x, table)
    return _mlp_tc(pooled, W1, b1, W2, b2)
